# baseline (speedup 1.0000x reference)
import functools

import jax
import jax.numpy as jnp
from jax.experimental import pallas as pl
from jax.experimental.pallas import tpu as pltpu


def _gcn_body(x_ref, norm_ref, W1_ref, b1_ref, W2_ref, b2_ref,
              out_ref, A_s, B_s, *, BM):
    p = pl.program_id(0)
    i = pl.program_id(1)

    @pl.when(jnp.logical_and(p == 0, i == 0))
    def _():
        A_s[:] = jnp.dot(x_ref[:], W1_ref[:],
                         preferred_element_type=jnp.float32)

    @pl.when(p == 0)
    def _():
        acc = jnp.dot(norm_ref[:], A_s[:],
                      preferred_element_type=jnp.float32)
        h_blk = jnp.maximum(acc + b1_ref[:], 0.0)
        B_s[pl.ds(i * BM, BM), :] = jnp.dot(h_blk, W2_ref[:],
                                            preferred_element_type=jnp.float32)

    @pl.when(p == 1)
    def _():
        z = jnp.dot(norm_ref[:], B_s[:],
                    preferred_element_type=jnp.float32) + b2_ref[:]
        m = jnp.max(z, axis=1, keepdims=True)
        e = jnp.exp(z - m)
        out_ref[:] = jnp.log(e / jnp.sum(e, axis=1, keepdims=True))


def kernel(x, norm, W1, b1, W2, b2):
    N, F_IN = x.shape
    HID = W1.shape[1]
    C = W2.shape[1]
    BM = 200
    NB = N // BM

    out = pl.pallas_call(
        functools.partial(_gcn_body, BM=BM),
        grid=(2, NB),
        in_specs=[
            pl.BlockSpec((N, F_IN), lambda p, i: (0, 0)),
            pl.BlockSpec((BM, N), lambda p, i: (i, 0)),
            pl.BlockSpec((F_IN, HID), lambda p, i: (0, 0)),
            pl.BlockSpec((1, HID), lambda p, i: (0, 0)),
            pl.BlockSpec((HID, C), lambda p, i: (0, 0)),
            pl.BlockSpec((1, C), lambda p, i: (0, 0)),
        ],
        out_specs=pl.BlockSpec((BM, C), lambda p, i: (i * p, 0)),
        out_shape=jax.ShapeDtypeStruct((N, C), jnp.float32),
        scratch_shapes=[
            pltpu.VMEM((N, HID), jnp.float32),
            pltpu.VMEM((N, C), jnp.float32),
        ],
        compiler_params=pltpu.CompilerParams(
            dimension_semantics=("arbitrary", "arbitrary"),
        ),
    )(x, norm, W1, b1.reshape(1, HID), W2, b2.reshape(1, C))
    return out


# speedup vs baseline: 1.0370x; 1.0370x over previous
import functools

import jax
import jax.numpy as jnp
from jax.experimental import pallas as pl
from jax.experimental.pallas import tpu as pltpu


def _gcn_body(x_ref, norm_ref, W1_ref, b1_ref, W2_ref, b2_ref,
              out_ref, A_s, B_s, *, BM):
    p = pl.program_id(0)
    i = pl.program_id(1)

    @pl.when(jnp.logical_and(p == 0, i == 0))
    def _():
        A_s[:] = jnp.dot(x_ref[:], W1_ref[:],
                         preferred_element_type=jnp.float32)

    @pl.when(p == 0)
    def _():
        acc = jnp.dot(norm_ref[:], A_s[:],
                      preferred_element_type=jnp.float32)
        h_blk = jnp.maximum(acc + b1_ref[:], 0.0)
        B_s[pl.ds(i * BM, BM), :] = jnp.dot(h_blk, W2_ref[:],
                                            preferred_element_type=jnp.float32)

    @pl.when(p == 1)
    def _():
        z = jnp.dot(norm_ref[:], B_s[:],
                    preferred_element_type=jnp.float32) + b2_ref[:]
        m = jnp.max(z, axis=1, keepdims=True)
        e = jnp.exp(z - m)
        out_ref[:] = jnp.log(e / jnp.sum(e, axis=1, keepdims=True))


def kernel(x, norm, W1, b1, W2, b2):
    N, F_IN = x.shape
    HID = W1.shape[1]
    C = W2.shape[1]
    BM = 400
    NB = N // BM

    out = pl.pallas_call(
        functools.partial(_gcn_body, BM=BM),
        grid=(2, NB),
        in_specs=[
            pl.BlockSpec((N, F_IN), lambda p, i: (0, 0)),
            pl.BlockSpec((BM, N), lambda p, i: (i, 0)),
            pl.BlockSpec((F_IN, HID), lambda p, i: (0, 0)),
            pl.BlockSpec((1, HID), lambda p, i: (0, 0)),
            pl.BlockSpec((HID, C), lambda p, i: (0, 0)),
            pl.BlockSpec((1, C), lambda p, i: (0, 0)),
        ],
        out_specs=pl.BlockSpec((BM, C), lambda p, i: (i * p, 0)),
        out_shape=jax.ShapeDtypeStruct((N, C), jnp.float32),
        scratch_shapes=[
            pltpu.VMEM((N, HID), jnp.float32),
            pltpu.VMEM((N, C), jnp.float32),
        ],
        compiler_params=pltpu.CompilerParams(
            dimension_semantics=("arbitrary", "arbitrary"),
        ),
    )(x, norm, W1, b1.reshape(1, HID), W2, b2.reshape(1, C))
    return out


# speedup vs baseline: 1.0415x; 1.0043x over previous
import functools

import jax
import jax.numpy as jnp
from jax.experimental import pallas as pl
from jax.experimental.pallas import tpu as pltpu


def _gcn_body(x_ref, norm_ref, W1_ref, b1_ref, W2_ref, b2_ref,
              out_ref, A_s, B_s, *, BM):
    p = pl.program_id(0)
    i = pl.program_id(1)

    @pl.when(jnp.logical_and(p == 0, i == 0))
    def _():
        A_s[:] = jnp.dot(x_ref[:], W1_ref[:],
                         preferred_element_type=jnp.float32)

    @pl.when(p == 0)
    def _():
        acc = jnp.dot(norm_ref[:], A_s[:],
                      preferred_element_type=jnp.float32)
        h_blk = jnp.maximum(acc + b1_ref[:], 0.0)
        B_s[pl.ds(i * BM, BM), :] = jnp.dot(h_blk, W2_ref[:],
                                            preferred_element_type=jnp.float32)

    @pl.when(p == 1)
    def _():
        z = jnp.dot(norm_ref[:], B_s[:],
                    preferred_element_type=jnp.float32) + b2_ref[:]
        m = jnp.max(z, axis=1, keepdims=True)
        e = jnp.exp(z - m)
        out_ref[:] = jnp.log(e / jnp.sum(e, axis=1, keepdims=True))


def kernel(x, norm, W1, b1, W2, b2):
    N, F_IN = x.shape
    HID = W1.shape[1]
    C = W2.shape[1]
    BM = 400
    NB = N // BM

    out = pl.pallas_call(
        functools.partial(_gcn_body, BM=BM),
        grid=(2, NB),
        in_specs=[
            pl.BlockSpec((N, F_IN), lambda p, i: (0, 0)),
            pl.BlockSpec((BM, N), lambda p, i: (i + p * (NB - 1 - 2 * i), 0)),
            pl.BlockSpec((F_IN, HID), lambda p, i: (0, 0)),
            pl.BlockSpec((1, HID), lambda p, i: (0, 0)),
            pl.BlockSpec((HID, C), lambda p, i: (0, 0)),
            pl.BlockSpec((1, C), lambda p, i: (0, 0)),
        ],
        out_specs=pl.BlockSpec((BM, C),
                               lambda p, i: (NB - 1 - p * i, 0)),
        out_shape=jax.ShapeDtypeStruct((N, C), jnp.float32),
        scratch_shapes=[
            pltpu.VMEM((N, HID), jnp.float32),
            pltpu.VMEM((N, C), jnp.float32),
        ],
        compiler_params=pltpu.CompilerParams(
            dimension_semantics=("arbitrary", "arbitrary"),
        ),
    )(x, norm, W1, b1.reshape(1, HID), W2, b2.reshape(1, C))
    return out
